# Initial kernel scaffold; baseline (speedup 1.0000x reference)
import functools

import jax
import jax.numpy as jnp
import numpy as np
from jax import lax
from jax.experimental import pallas as pl
from jax.experimental.pallas import tpu as pltpu
from jax.experimental.pallas import tpu_sc as plsc

N = 10000
E = 320000
EMB = 128
HID = 256
NE = 3
NL = 2
BN_EPS = 1e-5

NC, NS = 2, 16
CHUNK = 128
E_PAD = 323584
CPT = E_PAD // (NC * NS * CHUNK)
RPT = N // NS
ACC_ROWS = N + 8

NB = 2000
NBLK = N // NB


def _prep_body(xf_ref, ad_ref, ab_ref, a0_ref, a1_ref, a2_ref, h0_ref, t_ref):
    h0_ref[...] = (
        jnp.dot(xf_ref[...], ad_ref[...], preferred_element_type=jnp.float32)
        + ab_ref[...]
    )
    t_ref[...] = a0_ref[...] + 2 * a1_ref[...] + 4 * a2_ref[...]


def _prep(xf, adiff, abase, a0, a1, a2):
    return pl.pallas_call(
        _prep_body,
        out_shape=[
            jax.ShapeDtypeStruct((N, EMB), jnp.float32),
            jax.ShapeDtypeStruct((E // 128, 128), jnp.int32),
        ],
    )(xf, adiff, abase, a0, a1, a2)


def _sc_layer_body(h_hbm, src_hbm, dst_hbm, t_hbm, etab_hbm, zrows_hbm,
                   out_hbm, src_v, dst_v, t_v, rows_v, etab_v, acc_sh,
                   etab_sh, sem):
    cid = lax.axis_index("c")
    sid = lax.axis_index("s")

    for e in range(NE):
        @pl.when(sid == 0)
        def _():
            pltpu.sync_copy(etab_hbm.at[e], etab_v)
            pltpu.sync_copy(etab_v, etab_sh)

        pltpu.sync_copy(zrows_hbm, acc_sh.at[pl.ds(sid * RPT, RPT)])
        plsc.subcore_barrier()

        def chunk_body(ci, _):
            base = ((sid * NC + cid) * CPT + ci) * CHUNK
            pltpu.sync_copy(src_hbm.at[pl.ds(base, CHUNK)], src_v)
            pltpu.sync_copy(t_hbm.at[pl.ds(base, CHUNK)], t_v)
            pltpu.sync_copy(dst_hbm.at[pl.ds(base, CHUNK)], dst_v)
            pltpu.async_copy(h_hbm.at[src_v], rows_v, sem).wait()
            pltpu.sync_copy(etab_sh.at[t_v], rows_v, add=True)

            def row_body(r, _):
                for k in range(EMB // 16):
                    sl = pl.ds(k * 16, 16)
                    rows_v[r, sl] = jnp.maximum(rows_v[r, sl], 0.0)
                return 0

            lax.fori_loop(0, CHUNK, row_body, 0)
            pltpu.sync_copy(rows_v, acc_sh.at[dst_v], add=True)
            return 0

        lax.fori_loop(0, CPT, chunk_body, 0)
        plsc.subcore_barrier()
        pltpu.sync_copy(
            acc_sh.at[pl.ds(sid * RPT, RPT)],
            out_hbm.at[e, cid, pl.ds(sid * RPT, RPT)],
        )


def _sc_layer(h, srcp, dstp, tp, etab_l, zrows):
    f = pl.kernel(
        _sc_layer_body,
        out_type=jax.ShapeDtypeStruct((NE, NC, N, EMB), jnp.float32),
        mesh=plsc.VectorSubcoreMesh(core_axis_name="c", subcore_axis_name="s"),
        scratch_types=[
            pltpu.VMEM((CHUNK,), jnp.int32),
            pltpu.VMEM((CHUNK,), jnp.int32),
            pltpu.VMEM((CHUNK,), jnp.int32),
            pltpu.VMEM((CHUNK, EMB), jnp.float32),
            pltpu.VMEM((8, EMB), jnp.float32),
            pltpu.VMEM_SHARED((ACC_ROWS, EMB), jnp.float32),
            pltpu.VMEM_SHARED((8, EMB), jnp.float32),
            pltpu.SemaphoreType.DMA,
        ],
    )
    return f(h, srcp, dstp, tp, etab_l, zrows)


def _p1_body(h_ref, p_ref, w_ref, b_ref, eps_ref, z1_ref, s1_ref, acc):
    nb = pl.program_id(1)
    u = (1.0 + eps_ref[0, 0]) * h_ref[...] + p_ref[0, 0] + p_ref[0, 1]
    z = jnp.dot(u, w_ref[0], preferred_element_type=jnp.float32) + b_ref[...]
    z1_ref[0] = z
    cs = jnp.sum(z, axis=0)
    cq = jnp.sum(z * z, axis=0)
    blk = jnp.stack([cs, cq], axis=0)

    @pl.when(nb == 0)
    def _():
        acc[...] = blk

    @pl.when(nb > 0)
    def _():
        acc[...] = acc[...] + blk

    s1_ref[0] = acc[...]


def _p1(h, P, W1l, b1l, epsl):
    return pl.pallas_call(
        _p1_body,
        grid=(NE, NBLK),
        in_specs=[
            pl.BlockSpec((NB, EMB), lambda e, n: (n, 0)),
            pl.BlockSpec((1, NC, NB, EMB), lambda e, n: (e, 0, n, 0)),
            pl.BlockSpec((1, EMB, HID), lambda e, n: (e, 0, 0)),
            pl.BlockSpec((1, HID), lambda e, n: (e, 0)),
            pl.BlockSpec((1, 1), lambda e, n: (e, 0)),
        ],
        out_specs=[
            pl.BlockSpec((1, NB, HID), lambda e, n: (e, n, 0)),
            pl.BlockSpec((1, 2, HID), lambda e, n: (e, 0, 0)),
        ],
        out_shape=[
            jax.ShapeDtypeStruct((NE, N, HID), jnp.float32),
            jax.ShapeDtypeStruct((NE, 2, HID), jnp.float32),
        ],
        scratch_shapes=[pltpu.VMEM((2, HID), jnp.float32)],
    )(h, P, W1l, b1l, epsl)


def _p2_body(z1_ref, s1_ref, g_ref, be_ref, w_ref, b_ref, z2_ref, s2_ref, acc):
    nb = pl.program_id(1)
    mu = s1_ref[0, 0] / N
    var = s1_ref[0, 1] / N - mu * mu
    inv = lax.rsqrt(var + BN_EPS)
    a = jnp.maximum((z1_ref[0] - mu) * (inv * g_ref[0]) + be_ref[0], 0.0)
    z = jnp.dot(a, w_ref[0], preferred_element_type=jnp.float32) + b_ref[...]
    z2_ref[0] = z
    cs = jnp.sum(z, axis=0)
    cq = jnp.sum(z * z, axis=0)
    blk = jnp.stack([cs, cq], axis=0)

    @pl.when(nb == 0)
    def _():
        acc[...] = blk

    @pl.when(nb > 0)
    def _():
        acc[...] = acc[...] + blk

    s2_ref[0] = acc[...]


def _p2(z1, s1, g1l, be1l, W2l, b2l):
    return pl.pallas_call(
        _p2_body,
        grid=(NE, NBLK),
        in_specs=[
            pl.BlockSpec((1, NB, HID), lambda e, n: (e, n, 0)),
            pl.BlockSpec((1, 2, HID), lambda e, n: (e, 0, 0)),
            pl.BlockSpec((1, HID), lambda e, n: (e, 0)),
            pl.BlockSpec((1, HID), lambda e, n: (e, 0)),
            pl.BlockSpec((1, HID, EMB), lambda e, n: (e, 0, 0)),
            pl.BlockSpec((1, EMB), lambda e, n: (e, 0)),
        ],
        out_specs=[
            pl.BlockSpec((1, NB, EMB), lambda e, n: (e, n, 0)),
            pl.BlockSpec((1, 2, EMB), lambda e, n: (e, 0, 0)),
        ],
        out_shape=[
            jax.ShapeDtypeStruct((NE, N, EMB), jnp.float32),
            jax.ShapeDtypeStruct((NE, 2, EMB), jnp.float32),
        ],
        scratch_shapes=[pltpu.VMEM((2, EMB), jnp.float32)],
    )(z1, s1, g1l, be1l, W2l, b2l)


def _p3_body(z2_ref, s2_ref, g_ref, be_ref, out_ref, *, apply_relu):
    acc = jnp.zeros((NB, EMB), jnp.float32)
    for e in range(NE):
        mu = s2_ref[e, 0] / N
        var = s2_ref[e, 1] / N - mu * mu
        inv = lax.rsqrt(var + BN_EPS)
        acc = acc + (z2_ref[e] - mu) * (inv * g_ref[e]) + be_ref[e]
    acc = acc * (1.0 / NE)
    if apply_relu:
        acc = jnp.maximum(acc, 0.0)
    out_ref[...] = acc


def _p3(z2, s2, g2l, be2l, apply_relu):
    return pl.pallas_call(
        functools.partial(_p3_body, apply_relu=apply_relu),
        grid=(NBLK,),
        in_specs=[
            pl.BlockSpec((NE, NB, EMB), lambda n: (0, n, 0)),
            pl.BlockSpec((NE, 2, EMB), lambda n: (0, 0, 0)),
            pl.BlockSpec((NE, EMB), lambda n: (0, 0)),
            pl.BlockSpec((NE, EMB), lambda n: (0, 0)),
        ],
        out_specs=pl.BlockSpec((NB, EMB), lambda n: (n, 0)),
        out_shape=jax.ShapeDtypeStruct((N, EMB), jnp.float32),
    )(z2, s2, g2l, be2l)


def kernel(x, edge_index, edge_attr, batch, atom_tables, bond_tables,
           W1, b1, g1, be1, W2, b2, eps_p, g2, be2):
    xf = jnp.pad(x.astype(jnp.float32), ((0, 0), (0, 7)))
    adiff = jnp.pad(atom_tables[:, 1] - atom_tables[:, 0], ((0, 7), (0, 0)))
    abase = jnp.sum(atom_tables[:, 0], axis=0, keepdims=True)
    a0 = edge_attr[:, 0].reshape(E // 128, 128)
    a1 = edge_attr[:, 1].reshape(E // 128, 128)
    a2 = edge_attr[:, 2].reshape(E // 128, 128)

    h0, t2d = _prep(xf, adiff, abase, a0, a1, a2)
    t = t2d.reshape(E)

    npad = E_PAD - E
    srcp = jnp.concatenate([edge_index[0], jnp.zeros((npad,), jnp.int32)])
    dstp = jnp.concatenate([edge_index[1], jnp.full((npad,), N, jnp.int32)])
    tp = jnp.concatenate([t, jnp.zeros((npad,), jnp.int32)])
    zrows = jnp.zeros((RPT, EMB), jnp.float32)

    bits = np.stack([np.arange(8) & 1, (np.arange(8) >> 1) & 1,
                     (np.arange(8) >> 2) & 1], axis=1)
    Etab = (bond_tables[:, :, 0, bits[:, 0], :]
            + bond_tables[:, :, 1, bits[:, 1], :]
            + bond_tables[:, :, 2, bits[:, 2], :])

    h = h0
    for layer in range(NL):
        P = _sc_layer(h, srcp, dstp, tp, Etab[layer], zrows)
        z1, s1 = _p1(h, P, W1[layer], b1[layer],
                     eps_p[layer].reshape(NE, 1))
        z2, s2 = _p2(z1, s1, g1[layer], be1[layer], W2[layer], b2[layer])
        h = _p3(z2, s2, g2[layer], be2[layer], apply_relu=(layer < NL - 1))
    return h


# speedup vs baseline: 4.4391x; 4.4391x over previous
import functools

import jax
import jax.numpy as jnp
import numpy as np
from jax import lax
from jax.experimental import pallas as pl
from jax.experimental.pallas import tpu as pltpu
from jax.experimental.pallas import tpu_sc as plsc

N = 10000
E = 320000
EMB = 128
HID = 256
NE = 3
NL = 2
BN_EPS = 1e-5

NC, NS = 2, 16
CHUNK = 128
E_PAD = 323584
CPT = E_PAD // (NC * NS * CHUNK)
ACC_ROWS = 10240
RPT = ACC_ROWS // NS

NB = 2000
NBLK = N // NB


def _prep_body(xf_ref, ad_ref, ab_ref, a0_ref, a1_ref, a2_ref, h0_ref, t_ref):
    h0_ref[...] = (
        jnp.dot(xf_ref[...], ad_ref[...], preferred_element_type=jnp.float32, precision=lax.Precision.HIGHEST)
        + ab_ref[...]
    )
    t_ref[...] = a0_ref[...] + 2 * a1_ref[...] + 4 * a2_ref[...]


def _prep(xf, adiff, abase, a0, a1, a2):
    return pl.pallas_call(
        _prep_body,
        out_shape=[
            jax.ShapeDtypeStruct((N, EMB), jnp.float32),
            jax.ShapeDtypeStruct((E // 128, 128), jnp.int32),
        ],
    )(xf, adiff, abase, a0, a1, a2)


def _sc_layer_body(h_hbm, src_hbm, dst_hbm, t_hbm, etab_hbm, zrows_hbm,
                   out_hbm, src_v, dst_v, t_v, rows_v, etab_v, acc_sh,
                   etab_sh, sem):
    cid = lax.axis_index("c")
    sid = lax.axis_index("s")

    for e in range(NE):
        @pl.when(sid == 0)
        def _():
            pltpu.sync_copy(etab_hbm.at[e], etab_v)
            pltpu.sync_copy(etab_v, etab_sh)

        pltpu.sync_copy(zrows_hbm, acc_sh.at[pl.ds(sid * RPT, RPT)])
        plsc.subcore_barrier()

        def chunk_body(ci, _):
            base = ((sid * NC + cid) * CPT + ci) * CHUNK
            pltpu.sync_copy(src_hbm.at[pl.ds(base, CHUNK)], src_v)
            pltpu.sync_copy(t_hbm.at[pl.ds(base, CHUNK)], t_v)
            pltpu.sync_copy(dst_hbm.at[pl.ds(base, CHUNK)], dst_v)
            pltpu.async_copy(h_hbm.at[src_v], rows_v, sem).wait()
            pltpu.sync_copy(etab_sh.at[t_v], rows_v, add=True)

            def row_body(r, _):
                for k in range(EMB // 16):
                    sl = pl.ds(k * 16, 16)
                    rows_v[r, sl] = jnp.maximum(rows_v[r, sl], 0.0)
                return 0

            lax.fori_loop(0, CHUNK, row_body, 0)
            pltpu.sync_copy(rows_v, acc_sh.at[dst_v], add=True)
            return 0

        lax.fori_loop(0, CPT, chunk_body, 0)
        plsc.subcore_barrier()
        pltpu.sync_copy(
            acc_sh.at[pl.ds(sid * RPT, RPT)],
            out_hbm.at[e, cid, pl.ds(sid * RPT, RPT)],
        )


def _sc_layer(h, srcp, dstp, tp, etab_l, zrows):
    f = pl.kernel(
        _sc_layer_body,
        out_type=jax.ShapeDtypeStruct((NE, NC, ACC_ROWS, EMB), jnp.float32),
        mesh=plsc.VectorSubcoreMesh(core_axis_name="c", subcore_axis_name="s"),
        scratch_types=[
            pltpu.VMEM((CHUNK,), jnp.int32),
            pltpu.VMEM((CHUNK,), jnp.int32),
            pltpu.VMEM((CHUNK,), jnp.int32),
            pltpu.VMEM((CHUNK, EMB), jnp.float32),
            pltpu.VMEM((8, EMB), jnp.float32),
            pltpu.VMEM_SHARED((ACC_ROWS, EMB), jnp.float32),
            pltpu.VMEM_SHARED((8, EMB), jnp.float32),
            pltpu.SemaphoreType.DMA,
        ],
    )
    return f(h, srcp, dstp, tp, etab_l, zrows)


def _p1_body(h_ref, p_ref, w_ref, b_ref, eps_ref, z1_ref, s1_ref, acc):
    nb = pl.program_id(1)
    u = (1.0 + eps_ref[0, 0, 0]) * h_ref[...] + p_ref[0, 0] + p_ref[0, 1]
    z = jnp.dot(u, w_ref[0], preferred_element_type=jnp.float32) + b_ref[0]
    z1_ref[0] = z
    cs = jnp.sum(z, axis=0)
    cq = jnp.sum(z * z, axis=0)
    blk = jnp.stack([cs, cq], axis=0)

    @pl.when(nb == 0)
    def _():
        acc[...] = blk

    @pl.when(nb > 0)
    def _():
        acc[...] = acc[...] + blk

    s1_ref[0] = acc[...]


def _p1(h, P, W1l, b1l, epsl):
    return pl.pallas_call(
        _p1_body,
        grid=(NE, NBLK),
        in_specs=[
            pl.BlockSpec((NB, EMB), lambda e, n: (n, 0)),
            pl.BlockSpec((1, NC, NB, EMB), lambda e, n: (e, 0, n, 0)),
            pl.BlockSpec((1, EMB, HID), lambda e, n: (e, 0, 0)),
            pl.BlockSpec((1, 1, HID), lambda e, n: (e, 0, 0)),
            pl.BlockSpec((1, 1, 1), lambda e, n: (e, 0, 0)),
        ],
        out_specs=[
            pl.BlockSpec((1, NB, HID), lambda e, n: (e, n, 0)),
            pl.BlockSpec((1, 2, HID), lambda e, n: (e, 0, 0)),
        ],
        out_shape=[
            jax.ShapeDtypeStruct((NE, N, HID), jnp.float32),
            jax.ShapeDtypeStruct((NE, 2, HID), jnp.float32),
        ],
        scratch_shapes=[pltpu.VMEM((2, HID), jnp.float32)],
    )(h, P, W1l, b1l, epsl)


def _p2_body(z1_ref, s1_ref, g_ref, be_ref, w_ref, b_ref, z2_ref, s2_ref, acc):
    nb = pl.program_id(1)
    mu = s1_ref[0, 0] / N
    var = s1_ref[0, 1] / N - mu * mu
    inv = lax.rsqrt(var + BN_EPS)
    a = jnp.maximum((z1_ref[0] - mu) * (inv * g_ref[0, 0]) + be_ref[0, 0], 0.0)
    z = jnp.dot(a, w_ref[0], preferred_element_type=jnp.float32) + b_ref[0]
    z2_ref[0] = z
    cs = jnp.sum(z, axis=0)
    cq = jnp.sum(z * z, axis=0)
    blk = jnp.stack([cs, cq], axis=0)

    @pl.when(nb == 0)
    def _():
        acc[...] = blk

    @pl.when(nb > 0)
    def _():
        acc[...] = acc[...] + blk

    s2_ref[0] = acc[...]


def _p2(z1, s1, g1l, be1l, W2l, b2l):
    return pl.pallas_call(
        _p2_body,
        grid=(NE, NBLK),
        in_specs=[
            pl.BlockSpec((1, NB, HID), lambda e, n: (e, n, 0)),
            pl.BlockSpec((1, 2, HID), lambda e, n: (e, 0, 0)),
            pl.BlockSpec((1, 1, HID), lambda e, n: (e, 0, 0)),
            pl.BlockSpec((1, 1, HID), lambda e, n: (e, 0, 0)),
            pl.BlockSpec((1, HID, EMB), lambda e, n: (e, 0, 0)),
            pl.BlockSpec((1, 1, EMB), lambda e, n: (e, 0, 0)),
        ],
        out_specs=[
            pl.BlockSpec((1, NB, EMB), lambda e, n: (e, n, 0)),
            pl.BlockSpec((1, 2, EMB), lambda e, n: (e, 0, 0)),
        ],
        out_shape=[
            jax.ShapeDtypeStruct((NE, N, EMB), jnp.float32),
            jax.ShapeDtypeStruct((NE, 2, EMB), jnp.float32),
        ],
        scratch_shapes=[pltpu.VMEM((2, EMB), jnp.float32)],
    )(z1, s1, g1l, be1l, W2l, b2l)


def _p3_body(z2_ref, s2_ref, g_ref, be_ref, out_ref, *, apply_relu):
    acc = jnp.zeros((NB, EMB), jnp.float32)
    for e in range(NE):
        mu = s2_ref[e, 0] / N
        var = s2_ref[e, 1] / N - mu * mu
        inv = lax.rsqrt(var + BN_EPS)
        acc = acc + (z2_ref[e] - mu) * (inv * g_ref[e]) + be_ref[e]
    acc = acc * (1.0 / NE)
    if apply_relu:
        acc = jnp.maximum(acc, 0.0)
    out_ref[...] = acc


def _p3(z2, s2, g2l, be2l, apply_relu):
    return pl.pallas_call(
        functools.partial(_p3_body, apply_relu=apply_relu),
        grid=(NBLK,),
        in_specs=[
            pl.BlockSpec((NE, NB, EMB), lambda n: (0, n, 0)),
            pl.BlockSpec((NE, 2, EMB), lambda n: (0, 0, 0)),
            pl.BlockSpec((NE, EMB), lambda n: (0, 0)),
            pl.BlockSpec((NE, EMB), lambda n: (0, 0)),
        ],
        out_specs=pl.BlockSpec((NB, EMB), lambda n: (n, 0)),
        out_shape=jax.ShapeDtypeStruct((N, EMB), jnp.float32),
    )(z2, s2, g2l, be2l)


def kernel(x, edge_index, edge_attr, batch, atom_tables, bond_tables,
           W1, b1, g1, be1, W2, b2, eps_p, g2, be2):
    xf = jnp.pad(x.astype(jnp.float32), ((0, 0), (0, 7)))
    adiff = jnp.pad(atom_tables[:, 1] - atom_tables[:, 0], ((0, 7), (0, 0)))
    abase = jnp.sum(atom_tables[:, 0], axis=0, keepdims=True)
    a0 = edge_attr[:, 0].reshape(E // 128, 128)
    a1 = edge_attr[:, 1].reshape(E // 128, 128)
    a2 = edge_attr[:, 2].reshape(E // 128, 128)

    h0, t2d = _prep(xf, adiff, abase, a0, a1, a2)
    t = t2d.reshape(E)

    npad = E_PAD - E
    srcp = jnp.concatenate([edge_index[0], jnp.zeros((npad,), jnp.int32)])
    dstp = jnp.concatenate([edge_index[1], jnp.full((npad,), N, jnp.int32)])
    tp = jnp.concatenate([t, jnp.zeros((npad,), jnp.int32)])
    zrows = jnp.zeros((RPT, EMB), jnp.float32)

    bits = np.stack([np.arange(8) & 1, (np.arange(8) >> 1) & 1,
                     (np.arange(8) >> 2) & 1], axis=1)
    Etab = (bond_tables[:, :, 0, bits[:, 0], :]
            + bond_tables[:, :, 1, bits[:, 1], :]
            + bond_tables[:, :, 2, bits[:, 2], :])

    h = h0
    for layer in range(NL):
        P = _sc_layer(h, srcp, dstp, tp, Etab[layer], zrows)
        z1, s1 = _p1(h, P, W1[layer], b1[layer].reshape(NE, 1, HID),
                     eps_p[layer].reshape(NE, 1, 1))
        z2, s2 = _p2(z1, s1, g1[layer].reshape(NE, 1, HID),
                     be1[layer].reshape(NE, 1, HID), W2[layer],
                     b2[layer].reshape(NE, 1, EMB))
        h = _p3(z2, s2, g2[layer], be2[layer], apply_relu=(layer < NL - 1))
    return h
